# Initial kernel scaffold; baseline (speedup 1.0000x reference)
import jax
import jax.numpy as jnp
from jax.experimental import pallas as pl


def kernel(x, idx, means, stds):
    raise NotImplementedError("write your pallas kernel here")


# speedup vs baseline: 1.9353x; 1.9353x over previous
import functools

import jax
import jax.numpy as jnp
from jax import lax
from jax.experimental import pallas as pl
from jax.experimental.pallas import tpu as pltpu
from jax.experimental.pallas import tpu_sc as plsc

N, D, K = 65536, 512, 64
NC, NS, L = 2, 16, 16
NW = NC * NS
RPW = N // NW
CHUNK = 64
NCHUNK = RPW // CHUNK
G = K // L


def _sc_body(x_hbm, idx_hbm, means_hbm, stds_hbm, out_hbm,
             buf, idx_v, m_v, s_v):
    wid = lax.axis_index("s") * NC + lax.axis_index("c")
    base = wid * RPW

    pltpu.sync_copy(idx_hbm, idx_v)
    pltpu.sync_copy(means_hbm, m_v)
    pltpu.sync_copy(stds_hbm, s_v)

    ci = [idx_v[pl.ds(g * L, L)] for g in range(G)]
    mm = [m_v[pl.ds(g * L, L)] for g in range(G)]
    inv = [1.0 / s_v[pl.ds(g * L, L)] for g in range(G)]

    def chunk_body(c, carry):
        row0 = base + c * CHUNK
        pltpu.sync_copy(x_hbm.at[pl.ds(row0, CHUNK)], buf)

        def row_body(r, rcarry):
            rs = jnp.full((L,), r, jnp.int32)
            for g in range(G):
                v = plsc.load_gather(buf, [rs, ci[g]])
                v = (v - mm[g]) * inv[g]
                plsc.store_scatter(buf, [rs, ci[g]], v)
            return rcarry

        lax.fori_loop(0, CHUNK, row_body, 0, unroll=False)
        pltpu.sync_copy(buf, out_hbm.at[pl.ds(row0, CHUNK)])
        return carry

    lax.fori_loop(0, NCHUNK, chunk_body, 0, unroll=False)


@jax.jit
def kernel(x, idx, means, stds):
    idx = idx.astype(jnp.int32)
    mesh = plsc.VectorSubcoreMesh(core_axis_name="c", subcore_axis_name="s")
    f = pl.kernel(
        _sc_body,
        out_type=jax.ShapeDtypeStruct((N, D), jnp.float32),
        mesh=mesh,
        compiler_params=pltpu.CompilerParams(needs_layout_passes=False),
        scratch_types=[
            pltpu.VMEM((CHUNK, D), jnp.float32),
            pltpu.VMEM((K,), jnp.int32),
            pltpu.VMEM((K,), jnp.float32),
            pltpu.VMEM((K,), jnp.float32),
        ],
    )
    return f(x, idx, means, stds)


# speedup vs baseline: 3.0193x; 1.5601x over previous
import jax
import jax.numpy as jnp
from jax import lax
from jax.experimental import pallas as pl
from jax.experimental.pallas import tpu as pltpu
from jax.experimental.pallas import tpu_sc as plsc

N, D, K = 65536, 512, 64
NC, NS, L = 2, 16, 16
NW = NC * NS
RPW = N // NW
CHUNK = 64
NCHUNK = RPW // CHUNK
NBUF = 2
NOUTER = NCHUNK // NBUF
G = K // L


def _sc_body(x_hbm, idx_hbm, means_hbm, stds_hbm, out_hbm,
             buf0, buf1, idx_v, m_v, s_v,
             isem0, isem1, osem0, osem1):
    bufs = (buf0, buf1)
    isems = (isem0, isem1)
    osems = (osem0, osem1)

    wid = lax.axis_index("s") * NC + lax.axis_index("c")
    base = wid * RPW

    pltpu.sync_copy(idx_hbm, idx_v)
    pltpu.sync_copy(means_hbm, m_v)
    pltpu.sync_copy(stds_hbm, s_v)

    ci = [idx_v[pl.ds(g * L, L)] for g in range(G)]
    mm = [m_v[pl.ds(g * L, L)] for g in range(G)]
    inv = [1.0 / s_v[pl.ds(g * L, L)] for g in range(G)]

    def fixup(buf):
        def row_body(r, rcarry):
            rs = jnp.full((L,), r, jnp.int32)
            for g in range(G):
                v = plsc.load_gather(buf, [rs, ci[g]])
                v = (v - mm[g]) * inv[g]
                plsc.store_scatter(buf, [rs, ci[g]], v)
            return rcarry
        lax.fori_loop(0, CHUNK, row_body, 0, unroll=False)

    for b in range(NBUF):
        pltpu.async_copy(x_hbm.at[pl.ds(base + b * CHUNK, CHUNK)],
                         bufs[b], isems[b])

    def outer(o, carry):
        for b in range(NBUF):
            c = o * NBUF + b
            row0 = base + c * CHUNK
            pltpu.make_async_copy(x_hbm.at[pl.ds(row0, CHUNK)],
                                  bufs[b], isems[b]).wait()
            fixup(bufs[b])
            pltpu.async_copy(bufs[b], out_hbm.at[pl.ds(row0, CHUNK)],
                             osems[b])

            @pl.when(c + NBUF < NCHUNK)
            def _():
                pltpu.make_async_copy(bufs[b],
                                      out_hbm.at[pl.ds(row0, CHUNK)],
                                      osems[b]).wait()
                pltpu.async_copy(
                    x_hbm.at[pl.ds(row0 + NBUF * CHUNK, CHUNK)],
                    bufs[b], isems[b])
        return carry

    lax.fori_loop(0, NOUTER, outer, 0, unroll=False)

    for b in range(NBUF):
        c = NCHUNK - NBUF + b
        pltpu.make_async_copy(bufs[b],
                              out_hbm.at[pl.ds(base + c * CHUNK, CHUNK)],
                              osems[b]).wait()


@jax.jit
def kernel(x, idx, means, stds):
    idx = idx.astype(jnp.int32)
    mesh = plsc.VectorSubcoreMesh(core_axis_name="c", subcore_axis_name="s")
    f = pl.kernel(
        _sc_body,
        out_type=jax.ShapeDtypeStruct((N, D), jnp.float32),
        mesh=mesh,
        compiler_params=pltpu.CompilerParams(needs_layout_passes=False),
        scratch_types=[
            pltpu.VMEM((CHUNK, D), jnp.float32),
            pltpu.VMEM((CHUNK, D), jnp.float32),
            pltpu.VMEM((K,), jnp.int32),
            pltpu.VMEM((K,), jnp.float32),
            pltpu.VMEM((K,), jnp.float32),
            pltpu.SemaphoreType.DMA,
            pltpu.SemaphoreType.DMA,
            pltpu.SemaphoreType.DMA,
            pltpu.SemaphoreType.DMA,
        ],
    )
    return f(x, idx, means, stds)
